# baseline (speedup 1.0000x reference)
import functools

import jax
import jax.numpy as jnp
from jax.experimental import pallas as pl
from jax.experimental.pallas import tpu as pltpu

SCALING = 2.0

BN_FOLD = 256
BM = 512


def _fold_kernel(q_ref, b_ref, a_ref, w_ref):
    w_ref[...] = q_ref[...] + SCALING * jax.lax.dot_general(
        b_ref[...], a_ref[...],
        dimension_numbers=(((1,), (0,)), ((), ())),
        preferred_element_type=jnp.float32,
    )


def _gemm_kernel(x_ref, w_ref, bias_ref, o_ref):
    acc = jax.lax.dot_general(
        x_ref[...], w_ref[...],
        dimension_numbers=(((1,), (1,)), ((), ())),
        preferred_element_type=jnp.float32,
    )
    o_ref[...] = acc + bias_ref[...]


@jax.jit
def kernel(x, quantized_weight, lora_A, lora_B, bias):
    n_out, n_in = quantized_weight.shape
    m = x.shape[0]

    w_eff = pl.pallas_call(
        _fold_kernel,
        grid=(n_out // BN_FOLD,),
        in_specs=[
            pl.BlockSpec((BN_FOLD, n_in), lambda i: (i, 0)),
            pl.BlockSpec((BN_FOLD, lora_A.shape[0]), lambda i: (i, 0)),
            pl.BlockSpec((lora_A.shape[0], n_in), lambda i: (0, 0)),
        ],
        out_specs=pl.BlockSpec((BN_FOLD, n_in), lambda i: (i, 0)),
        out_shape=jax.ShapeDtypeStruct((n_out, n_in), jnp.float32),
    )(quantized_weight, lora_B, lora_A)

    bias2d = bias.reshape(1, n_out)
    out = pl.pallas_call(
        _gemm_kernel,
        grid=(m // BM,),
        in_specs=[
            pl.BlockSpec((BM, n_in), lambda i: (i, 0)),
            pl.BlockSpec((n_out, n_in), lambda i: (0, 0)),
            pl.BlockSpec((1, n_out), lambda i: (0, 0)),
        ],
        out_specs=pl.BlockSpec((BM, n_out), lambda i: (i, 0)),
        out_shape=jax.ShapeDtypeStruct((m, n_out), jnp.float32),
        compiler_params=pltpu.CompilerParams(
            dimension_semantics=("parallel",),
        ),
    )(x, w_eff, bias2d)
    return out


# speedup vs baseline: 1.0708x; 1.0708x over previous
import jax
import jax.numpy as jnp
from jax.experimental import pallas as pl
from jax.experimental.pallas import tpu as pltpu

SCALING = 2.0

BM = 512


def _fused_kernel(x_ref, q_ref, b_ref, a_ref, bias_ref, o_ref, w_ref):
    @pl.when(pl.program_id(0) == 0)
    def _fold():
        w_ref[...] = q_ref[...] + SCALING * jax.lax.dot_general(
            b_ref[...], a_ref[...],
            dimension_numbers=(((1,), (0,)), ((), ())),
            preferred_element_type=jnp.float32,
        )

    acc = jax.lax.dot_general(
        x_ref[...], w_ref[...],
        dimension_numbers=(((1,), (1,)), ((), ())),
        preferred_element_type=jnp.float32,
    )
    o_ref[...] = acc + bias_ref[...]


@jax.jit
def kernel(x, quantized_weight, lora_A, lora_B, bias):
    n_out, n_in = quantized_weight.shape
    rank = lora_A.shape[0]
    m = x.shape[0]

    bias2d = bias.reshape(1, n_out)
    out = pl.pallas_call(
        _fused_kernel,
        grid=(m // BM,),
        in_specs=[
            pl.BlockSpec((BM, n_in), lambda i: (i, 0)),
            pl.BlockSpec((n_out, n_in), lambda i: (0, 0)),
            pl.BlockSpec((n_out, rank), lambda i: (0, 0)),
            pl.BlockSpec((rank, n_in), lambda i: (0, 0)),
            pl.BlockSpec((1, n_out), lambda i: (0, 0)),
        ],
        out_specs=pl.BlockSpec((BM, n_out), lambda i: (i, 0)),
        out_shape=jax.ShapeDtypeStruct((m, n_out), jnp.float32),
        scratch_shapes=[pltpu.VMEM((n_out, n_in), jnp.float32)],
        compiler_params=pltpu.CompilerParams(
            dimension_semantics=("arbitrary",),
        ),
    )(x, quantized_weight, lora_B, lora_A, bias2d)
    return out
